# baseline (speedup 1.0000x reference)
import jax
import jax.numpy as jnp
from jax.experimental import pallas as pl

_B, _CIN, _H, _W = 8, 384, 128, 128
_CP, _CO = 256, 80
_K, _STRIDE = 100, 4


def _nms_body(x_ref, o_ref):
    x = x_ref[0, 0]
    ninf_row = jnp.full((1, _W), -jnp.inf, dtype=x.dtype)
    up = jnp.concatenate([x[1:, :], ninf_row], axis=0)
    dn = jnp.concatenate([ninf_row, x[:-1, :]], axis=0)
    r = jnp.maximum(jnp.maximum(up, dn), x)
    ninf_col = jnp.full((_H, 1), -jnp.inf, dtype=x.dtype)
    lf = jnp.concatenate([r[:, 1:], ninf_col], axis=1)
    rt = jnp.concatenate([ninf_col, r[:, :-1]], axis=1)
    m = jnp.maximum(jnp.maximum(lf, rt), r)
    o_ref[0, 0] = jnp.where(m == x, x, 0.0)


def _nms(heat):
    return pl.pallas_call(
        _nms_body,
        grid=(_B, _CO),
        in_specs=[pl.BlockSpec((1, 1, _H, _W), lambda b, c: (b, c, 0, 0))],
        out_specs=pl.BlockSpec((1, 1, _H, _W), lambda b, c: (b, c, 0, 0)),
        out_shape=jax.ShapeDtypeStruct((_B, _CO, _H, _W), heat.dtype),
    )(heat)


def kernel(input, W1, b1, W2, b2):
    feat = jax.lax.conv_general_dilated(
        input, W1, window_strides=(1, 1), padding=[(1, 1), (1, 1)],
        dimension_numbers=("NCHW", "OIHW", "NCHW"))
    feat = jax.nn.relu(feat + b1[None, :, None, None])
    heat = jax.lax.conv_general_dilated(
        feat, W2, window_strides=(1, 1), padding=[(0, 0), (0, 0)],
        dimension_numbers=("NCHW", "OIHW", "NCHW"))
    heat = jax.nn.sigmoid(heat + b2[None, :, None, None])
    heat = _nms(heat)
    flat = heat.reshape(_B, -1)
    scores, idx = jax.lax.top_k(flat, _K)
    clses = idx // (_H * _W)
    rem = idx % (_H * _W)
    ys = (rem // _W) * _STRIDE
    xs = (rem % _W) * _STRIDE
    return scores, clses, ys, xs


# speedup vs baseline: 1.0052x; 1.0052x over previous
import jax
import jax.numpy as jnp
from jax.experimental import pallas as pl

_B, _CIN, _H, _W = 8, 384, 128, 128
_CP, _CO = 256, 80
_K, _STRIDE = 100, 4
_TR = 8
_NT = _H // _TR
_HP, _WP = 144, 136


def _conv_body(cur_ref, nxt_ref, w1_ref, w2_ref, b1_ref, b2_ref, o_ref):
    r16 = jnp.concatenate([cur_ref[0], nxt_ref[0]], axis=0)
    acc = jnp.zeros((_TR * _W, _CP), dtype=jnp.float32)
    for dy in range(3):
        a = r16[dy:dy + _TR]
        ady = jnp.concatenate([a[:, dx:dx + _W, :] for dx in range(3)], axis=2)
        acc = acc + jnp.dot(ady.reshape(_TR * _W, 3 * _CIN), w1_ref[dy],
                            preferred_element_type=jnp.float32)
    feat = jax.nn.relu(acc + b1_ref[0][None, :])
    h = jnp.dot(feat.astype(jnp.bfloat16), w2_ref[...],
                preferred_element_type=jnp.float32) + b2_ref[0][None, :]
    o_ref[0] = jax.nn.sigmoid(h)


def _conv_head(xp, w1c, w2m, b1, b2):
    return pl.pallas_call(
        _conv_body,
        grid=(_B, _NT),
        in_specs=[
            pl.BlockSpec((1, _TR, _WP, _CIN), lambda b, i: (b, i, 0, 0)),
            pl.BlockSpec((1, _TR, _WP, _CIN), lambda b, i: (b, i + 1, 0, 0)),
            pl.BlockSpec((3, 3 * _CIN, _CP), lambda b, i: (0, 0, 0)),
            pl.BlockSpec((_CP, _CO), lambda b, i: (0, 0)),
            pl.BlockSpec((1, _CP), lambda b, i: (0, 0)),
            pl.BlockSpec((1, _CO), lambda b, i: (0, 0)),
        ],
        out_specs=pl.BlockSpec((1, _TR * _W, _CO), lambda b, i: (b, i, 0)),
        out_shape=jax.ShapeDtypeStruct((_B, _H * _W, _CO), jnp.float32),
    )(xp, xp, w1c, w2m, b1, b2)


def _nms_body(x_ref, o_ref):
    x = x_ref[0, 0]
    ninf_row = jnp.full((1, _W), -jnp.inf, dtype=x.dtype)
    up = jnp.concatenate([x[1:, :], ninf_row], axis=0)
    dn = jnp.concatenate([ninf_row, x[:-1, :]], axis=0)
    r = jnp.maximum(jnp.maximum(up, dn), x)
    ninf_col = jnp.full((_H, 1), -jnp.inf, dtype=x.dtype)
    lf = jnp.concatenate([r[:, 1:], ninf_col], axis=1)
    rt = jnp.concatenate([ninf_col, r[:, :-1]], axis=1)
    m = jnp.maximum(jnp.maximum(lf, rt), r)
    o_ref[0, 0] = jnp.where(m == x, x, 0.0)


def _nms(heat):
    return pl.pallas_call(
        _nms_body,
        grid=(_B, _CO),
        in_specs=[pl.BlockSpec((1, 1, _H, _W), lambda b, c: (b, c, 0, 0))],
        out_specs=pl.BlockSpec((1, 1, _H, _W), lambda b, c: (b, c, 0, 0)),
        out_shape=jax.ShapeDtypeStruct((_B, _CO, _H, _W), jnp.float32),
    )(heat)


def kernel(input, W1, b1, W2, b2):
    xt = jnp.transpose(input, (0, 2, 3, 1)).astype(jnp.bfloat16)
    xp = jnp.pad(xt, ((0, 0), (1, _HP - _H - 1), (1, _WP - _W - 1), (0, 0)))
    w1c = jnp.stack([
        jnp.concatenate([W1[:, :, dy, dx].T for dx in range(3)], axis=0)
        for dy in range(3)]).astype(jnp.bfloat16)
    w2m = W2[:, :, 0, 0].T.astype(jnp.bfloat16)
    heat = _conv_head(xp, w1c, w2m, b1[None, :], b2[None, :])
    heat = jnp.transpose(heat.reshape(_B, _H, _W, _CO), (0, 3, 1, 2))
    heat = _nms(heat)
    flat = heat.reshape(_B, -1)
    scores, idx = jax.lax.top_k(flat, _K)
    clses = idx // (_H * _W)
    rem = idx % (_H * _W)
    ys = (rem // _W) * _STRIDE
    xs = (rem % _W) * _STRIDE
    return scores, clses, ys, xs


# speedup vs baseline: 2.6719x; 2.6580x over previous
import jax
import jax.numpy as jnp
from jax.experimental import pallas as pl

_B, _CIN, _H, _W = 8, 384, 128, 128
_CP, _CO = 256, 80
_K, _STRIDE = 100, 4
_TR = 8
_NT = _H // _TR
_HP, _WP = 144, 136


def _conv_body(cur_ref, nxt_ref, w1_ref, w2_ref, b1_ref, b2_ref, o_ref):
    r16 = jnp.concatenate([cur_ref[0], nxt_ref[0]], axis=0)
    acc = jnp.zeros((_TR * _W, _CP), dtype=jnp.float32)
    for dy in range(3):
        a = r16[dy:dy + _TR]
        ady = jnp.concatenate([a[:, dx:dx + _W, :] for dx in range(3)], axis=2)
        acc = acc + jnp.dot(ady.reshape(_TR * _W, 3 * _CIN), w1_ref[dy],
                            preferred_element_type=jnp.float32)
    feat = jax.nn.relu(acc + b1_ref[0][None, :])
    h = jnp.dot(feat.astype(jnp.bfloat16), w2_ref[...],
                preferred_element_type=jnp.float32) + b2_ref[0][None, :]
    o_ref[0] = jax.nn.sigmoid(h)


def _conv_head(xp, w1c, w2m, b1, b2):
    return pl.pallas_call(
        _conv_body,
        grid=(_B, _NT),
        in_specs=[
            pl.BlockSpec((1, _TR, _WP, _CIN), lambda b, i: (b, i, 0, 0)),
            pl.BlockSpec((1, _TR, _WP, _CIN), lambda b, i: (b, i + 1, 0, 0)),
            pl.BlockSpec((3, 3 * _CIN, _CP), lambda b, i: (0, 0, 0)),
            pl.BlockSpec((_CP, _CO), lambda b, i: (0, 0)),
            pl.BlockSpec((1, _CP), lambda b, i: (0, 0)),
            pl.BlockSpec((1, _CO), lambda b, i: (0, 0)),
        ],
        out_specs=pl.BlockSpec((1, _TR * _W, _CO), lambda b, i: (b, i, 0)),
        out_shape=jax.ShapeDtypeStruct((_B, _H * _W, _CO), jnp.float32),
    )(xp, xp, w1c, w2m, b1, b2)


_NP = _H * _W


def _shift(h, s):
    ninf = jnp.full((abs(s), _CO), -jnp.inf, dtype=h.dtype)
    if s > 0:
        return jnp.concatenate([h[s:], ninf], axis=0)
    return jnp.concatenate([ninf, h[:s]], axis=0)


def _decode_body(x_ref, sc_ref, id_ref, hn_ref, r_ref):
    h = x_ref[0]
    row = jax.lax.broadcasted_iota(jnp.int32, (_NP, _CO), 0)
    xgt0 = (row % _W) != 0
    xlt = (row % _W) != (_W - 1)
    ninf = jnp.float32(-jnp.inf)
    m = h
    for off, mask in ((-_W, None), (_W, None),
                      (-1, xgt0), (_W - 1, xgt0), (-_W - 1, xgt0),
                      (1, xlt), (-_W + 1, xlt), (_W + 1, xlt)):
        s = _shift(h, off)
        if mask is not None:
            s = jnp.where(mask, s, ninf)
        m = jnp.maximum(m, s)
    hn = jnp.where(h >= m, h, 0.0)
    hn_ref[...] = hn
    r_ref[...] = jnp.max(hn.reshape(_H, _W, _CO), axis=2)

    p2 = jax.lax.broadcasted_iota(jnp.int32, (_H, _W), 0) * _W \
        + jax.lax.broadcasted_iota(jnp.int32, (_H, _W), 1)
    li = jax.lax.broadcasted_iota(jnp.int32, (1, _CO), 1)
    ri = jax.lax.broadcasted_iota(jnp.int32, (1, _W), 1)
    ki = jax.lax.broadcasted_iota(jnp.int32, (1, 128), 1)
    big = jnp.int32(2**30)

    def body(k, carry):
        sc_v, id_v = carry
        rv = r_ref[...]
        mx = jnp.max(rv)
        p = jnp.min(jnp.where(rv == mx, p2, big))
        hrow = hn_ref[pl.ds(p, 1), :]
        mv = jnp.max(hrow)
        c = jnp.min(jnp.where(hrow == mv, li, big))
        flat = c * _NP + p
        sc_v = jnp.where(ki == k, mv, sc_v)
        id_v = jnp.where(ki == k, flat, id_v)
        hrow2 = jnp.where(li == c, -1.0, hrow)
        hn_ref[pl.ds(p, 1), :] = hrow2
        rrow = r_ref[pl.ds(p // _W, 1), :]
        rrow2 = jnp.where(ri == (p % _W), jnp.max(hrow2), rrow)
        r_ref[pl.ds(p // _W, 1), :] = rrow2
        return sc_v, id_v

    sc_v, id_v = jax.lax.fori_loop(
        0, _K, body,
        (jnp.zeros((1, 128), jnp.float32), jnp.zeros((1, 128), jnp.int32)))
    sc_ref[0] = sc_v
    id_ref[0] = id_v


def _decode(heat):
    from jax.experimental.pallas import tpu as pltpu
    return pl.pallas_call(
        _decode_body,
        grid=(_B,),
        in_specs=[pl.BlockSpec((1, _NP, _CO), lambda b: (b, 0, 0))],
        out_specs=[pl.BlockSpec((1, 1, 128), lambda b: (b, 0, 0)),
                   pl.BlockSpec((1, 1, 128), lambda b: (b, 0, 0))],
        out_shape=[jax.ShapeDtypeStruct((_B, 1, 128), jnp.float32),
                   jax.ShapeDtypeStruct((_B, 1, 128), jnp.int32)],
        scratch_shapes=[pltpu.VMEM((_NP, _CO), jnp.float32),
                        pltpu.VMEM((_H, _W), jnp.float32)],
    )(heat)


def kernel(input, W1, b1, W2, b2):
    xt = jnp.transpose(input, (0, 2, 3, 1)).astype(jnp.bfloat16)
    xp = jnp.pad(xt, ((0, 0), (1, _HP - _H - 1), (1, _WP - _W - 1), (0, 0)))
    w1c = jnp.stack([
        jnp.concatenate([W1[:, :, dy, dx].T for dx in range(3)], axis=0)
        for dy in range(3)]).astype(jnp.bfloat16)
    w2m = W2[:, :, 0, 0].T.astype(jnp.bfloat16)
    heat = _conv_head(xp, w1c, w2m, b1[None, :], b2[None, :])
    sc3, id3 = _decode(heat)
    scores = sc3[:, 0, :_K]
    idx = id3[:, 0, :_K]
    clses = idx // (_H * _W)
    rem = idx % (_H * _W)
    ys = (rem // _W) * _STRIDE
    xs = (rem % _W) * _STRIDE
    return scores, clses, ys, xs
